# baseline (speedup 1.0000x reference)
import functools

import jax
import jax.numpy as jnp
from jax import lax
from jax.experimental import pallas as pl
from jax.experimental.pallas import tpu as pltpu
from jax.experimental.pallas import tpu_sc as plsc

VOCAB = 100000
HIDDEN = 128
BATCH = 1024


_NC, _NS = 2, 16
_NW = _NC * _NS
_B_PER_W = BATCH // _NW


@functools.cache
def _make_sc_gather():
    mesh = plsc.VectorSubcoreMesh(core_axis_name="c", subcore_axis_name="s")

    @functools.partial(
        pl.kernel,
        mesh=mesh,
        out_type=jax.ShapeDtypeStruct((BATCH, HIDDEN), jnp.float32),
        scratch_types=[
            pltpu.VMEM((_B_PER_W,), jnp.int32),
            pltpu.VMEM((_B_PER_W, HIDDEN), jnp.float32),
            pltpu.SemaphoreType.DMA,
        ],
    )
    def sc_gather(table_hbm, idx_hbm, out_hbm, idx_v, rows_v, sem):
        wid = lax.axis_index("s") * _NC + lax.axis_index("c")
        base = wid * _B_PER_W
        pltpu.sync_copy(idx_hbm.at[pl.ds(base, _B_PER_W)], idx_v)
        pltpu.async_copy(table_hbm.at[idx_v], rows_v, sem).wait()
        pltpu.sync_copy(rows_v, out_hbm.at[pl.ds(base, _B_PER_W)])

    return sc_gather


_TILE_N = 512


def _proj_body(hidden_ref, w_ref, b_ref, out_ref):
    acc = lax.dot_general(
        hidden_ref[...], w_ref[...],
        dimension_numbers=(((1,), (1,)), ((), ())),
        preferred_element_type=jnp.float32,
    )
    out_ref[...] = acc + b_ref[...]


def _projection(hidden, lm_w, lm_b2d):
    grid = (pl.cdiv(VOCAB, _TILE_N),)
    return pl.pallas_call(
        _proj_body,
        grid=grid,
        in_specs=[
            pl.BlockSpec((BATCH, HIDDEN), lambda j: (0, 0)),
            pl.BlockSpec((_TILE_N, HIDDEN), lambda j: (j, 0)),
            pl.BlockSpec((1, _TILE_N), lambda j: (0, j)),
        ],
        out_specs=pl.BlockSpec((BATCH, _TILE_N), lambda j: (0, j)),
        out_shape=jax.ShapeDtypeStruct((BATCH, VOCAB), jnp.float32),
    )(hidden, lm_w, lm_b2d)


def kernel(input_ids, embed_table, lm_w, lm_b):
    hidden = jnp.take(embed_table, input_ids, axis=0)
    return _projection(hidden, lm_w, lm_b.reshape(1, VOCAB))


# speedup vs baseline: 1.1677x; 1.1677x over previous
import functools

import jax
import jax.numpy as jnp
from jax import lax
from jax.experimental import pallas as pl
from jax.experimental.pallas import tpu as pltpu
from jax.experimental.pallas import tpu_sc as plsc

VOCAB = 100000
HIDDEN = 128
BATCH = 1024


_NC, _NS = 2, 16
_NW = _NC * _NS
_B_PER_W = BATCH // _NW


@functools.cache
def _make_sc_gather():
    mesh = plsc.VectorSubcoreMesh(core_axis_name="c", subcore_axis_name="s")

    @functools.partial(
        pl.kernel,
        mesh=mesh,
        out_type=jax.ShapeDtypeStruct((BATCH, HIDDEN), jnp.float32),
        scratch_types=[
            pltpu.VMEM((_B_PER_W,), jnp.int32),
            pltpu.VMEM((_B_PER_W, HIDDEN), jnp.float32),
            pltpu.SemaphoreType.DMA,
        ],
    )
    def sc_gather(table_hbm, idx_hbm, out_hbm, idx_v, rows_v, sem):
        wid = lax.axis_index("s") * _NC + lax.axis_index("c")
        base = wid * _B_PER_W
        pltpu.sync_copy(idx_hbm.at[pl.ds(base, _B_PER_W)], idx_v)
        pltpu.async_copy(table_hbm.at[idx_v], rows_v, sem).wait()
        pltpu.sync_copy(rows_v, out_hbm.at[pl.ds(base, _B_PER_W)])

    return sc_gather


_TILE_N = 2048
_NBLK = pl.cdiv(VOCAB, _TILE_N)
_TAIL = VOCAB - (_NBLK - 1) * _TILE_N
_RING = 4


def _proj_body(hidden_ref, w_ref, b_ref, out_ref, scratch, tail, sems, tail_sem):
    j = pl.program_id(0)
    slot = lax.rem(j, _RING)

    @pl.when(j >= _RING)
    def _wait_prev():
        pltpu.make_async_copy(
            scratch.at[slot],
            out_ref.at[:, pl.ds((j - _RING) * _TILE_N, _TILE_N)],
            sems.at[slot],
        ).wait()

    acc = lax.dot_general(
        hidden_ref[...], w_ref[...],
        dimension_numbers=(((1,), (1,)), ((), ())),
        preferred_element_type=jnp.float32,
    )
    res = acc + b_ref[0]

    @pl.when(j < _NBLK - 1)
    def _start_full():
        scratch[slot] = res
        pltpu.make_async_copy(
            scratch.at[slot],
            out_ref.at[:, pl.ds(j * _TILE_N, _TILE_N)],
            sems.at[slot],
        ).start()

    @pl.when(j == _NBLK - 1)
    def _tail_and_drain():
        tail[...] = res[:, :_TAIL]
        pltpu.make_async_copy(
            tail,
            out_ref.at[:, pl.ds((_NBLK - 1) * _TILE_N, _TAIL)],
            tail_sem,
        ).start()
        for k in range(1, _RING):
            block = _NBLK - 1 - k
            kslot = block % _RING
            pltpu.make_async_copy(
                scratch.at[kslot],
                out_ref.at[:, pl.ds(block * _TILE_N, _TILE_N)],
                sems.at[kslot],
            ).wait()
        pltpu.make_async_copy(
            tail,
            out_ref.at[:, pl.ds((_NBLK - 1) * _TILE_N, _TAIL)],
            tail_sem,
        ).wait()


def _projection(hidden, lm_w, lm_b3d):
    return pl.pallas_call(
        _proj_body,
        grid=(_NBLK,),
        in_specs=[
            pl.BlockSpec((BATCH, HIDDEN), lambda j: (0, 0)),
            pl.BlockSpec((_TILE_N, HIDDEN), lambda j: (j, 0)),
            pl.BlockSpec((1, 1, _TILE_N), lambda j: (j, 0, 0)),
        ],
        out_specs=pl.BlockSpec(memory_space=pltpu.MemorySpace.HBM),
        out_shape=jax.ShapeDtypeStruct((BATCH, VOCAB), jnp.float32),
        scratch_shapes=[
            pltpu.VMEM((_RING, BATCH, _TILE_N), jnp.float32),
            pltpu.VMEM((BATCH, _TAIL), jnp.float32),
            pltpu.SemaphoreType.DMA((_RING,)),
            pltpu.SemaphoreType.DMA,
        ],
    )(hidden, lm_w, lm_b3d)


def kernel(input_ids, embed_table, lm_w, lm_b):
    hidden = _make_sc_gather()(embed_table, input_ids.astype(jnp.int32))
    lm_b_pad = jnp.pad(lm_b, (0, _NBLK * _TILE_N - VOCAB))
    return _projection(hidden, lm_w, lm_b_pad.reshape(_NBLK, 1, _TILE_N))


# speedup vs baseline: 2.6048x; 2.2307x over previous
import functools

import jax
import jax.numpy as jnp
from jax import lax
from jax.experimental import pallas as pl
from jax.experimental.pallas import tpu as pltpu
from jax.experimental.pallas import tpu_sc as plsc

VOCAB = 100000
HIDDEN = 128
BATCH = 1024


_NC, _NS = 2, 16
_NW = _NC * _NS
_B_PER_W = BATCH // _NW


@functools.cache
def _make_sc_gather():
    mesh = plsc.VectorSubcoreMesh(core_axis_name="c", subcore_axis_name="s")

    @functools.partial(
        pl.kernel,
        mesh=mesh,
        out_type=jax.ShapeDtypeStruct((BATCH, HIDDEN), jnp.float32),
        scratch_types=[
            pltpu.VMEM((_B_PER_W,), jnp.int32),
            pltpu.VMEM((_B_PER_W, HIDDEN), jnp.float32),
            pltpu.SemaphoreType.DMA,
        ],
    )
    def sc_gather(table_hbm, idx_hbm, out_hbm, idx_v, rows_v, sem):
        wid = lax.axis_index("s") * _NC + lax.axis_index("c")
        base = wid * _B_PER_W
        pltpu.sync_copy(idx_hbm.at[pl.ds(base, _B_PER_W)], idx_v)
        pltpu.async_copy(table_hbm.at[idx_v], rows_v, sem).wait()
        pltpu.sync_copy(rows_v, out_hbm.at[pl.ds(base, _B_PER_W)])

    return sc_gather


_TILE_V = 2048
_NBLK = pl.cdiv(VOCAB, _TILE_V)


def _proj_body(hidden_ref, w_ref, b_ref, out_ref):
    acc = lax.dot_general(
        w_ref[...], hidden_ref[...],
        dimension_numbers=(((1,), (1,)), ((), ())),
        preferred_element_type=jnp.float32,
    )
    out_ref[...] = acc + b_ref[0]


def _projection(hidden, lm_w, lm_b3d):
    return pl.pallas_call(
        _proj_body,
        grid=(_NBLK,),
        in_specs=[
            pl.BlockSpec((BATCH, HIDDEN), lambda j: (0, 0)),
            pl.BlockSpec((_TILE_V, HIDDEN), lambda j: (j, 0)),
            pl.BlockSpec((1, _TILE_V, 1), lambda j: (j, 0, 0)),
        ],
        out_specs=pl.BlockSpec((_TILE_V, BATCH), lambda j: (j, 0)),
        out_shape=jax.ShapeDtypeStruct((VOCAB, BATCH), jnp.float32),
    )(hidden, lm_w, lm_b3d)


def kernel(input_ids, embed_table, lm_w, lm_b):
    hidden = _make_sc_gather()(embed_table, input_ids.astype(jnp.int32))
    lm_b_pad = jnp.pad(lm_b, (0, _NBLK * _TILE_V - VOCAB))
    logits_t = _projection(hidden, lm_w, lm_b_pad.reshape(_NBLK, _TILE_V, 1))
    return logits_t.T


# speedup vs baseline: 3.6445x; 1.3991x over previous
import functools

import jax
import jax.numpy as jnp
from jax import lax
from jax.experimental import pallas as pl
from jax.experimental.pallas import tpu as pltpu
from jax.experimental.pallas import tpu_sc as plsc

VOCAB = 100000
HIDDEN = 128
BATCH = 1024


_NC, _NS = 2, 16
_NW = _NC * _NS
_B_PER_W = BATCH // _NW


@functools.cache
def _make_sc_gather():
    mesh = plsc.VectorSubcoreMesh(core_axis_name="c", subcore_axis_name="s")

    @functools.partial(
        pl.kernel,
        mesh=mesh,
        out_type=jax.ShapeDtypeStruct((BATCH, HIDDEN), jnp.float32),
        scratch_types=[
            pltpu.VMEM((_B_PER_W,), jnp.int32),
            pltpu.VMEM((_B_PER_W, HIDDEN), jnp.float32),
            pltpu.SemaphoreType.DMA,
        ],
    )
    def sc_gather(table_hbm, idx_hbm, out_hbm, idx_v, rows_v, sem):
        wid = lax.axis_index("s") * _NC + lax.axis_index("c")
        base = wid * _B_PER_W
        pltpu.sync_copy(idx_hbm.at[pl.ds(base, _B_PER_W)], idx_v)
        pltpu.async_copy(table_hbm.at[idx_v], rows_v, sem).wait()
        pltpu.sync_copy(rows_v, out_hbm.at[pl.ds(base, _B_PER_W)])

    return sc_gather


_TILE_V = 2048
_NBLK = pl.cdiv(VOCAB, _TILE_V)


def _proj_body(hidden_ref, w_ref, b_ref, out_ref):
    acc = lax.dot_general(
        w_ref[...], hidden_ref[...],
        dimension_numbers=(((1,), (1,)), ((), ())),
        preferred_element_type=jnp.float32,
    )
    out_ref[...] = acc + jnp.transpose(b_ref[0])


def _projection(hidden, lm_w, lm_b3d):
    return pl.pallas_call(
        _proj_body,
        grid=(_NBLK,),
        in_specs=[
            pl.BlockSpec((BATCH, HIDDEN), lambda j: (0, 0)),
            pl.BlockSpec((_TILE_V, HIDDEN), lambda j: (j, 0)),
            pl.BlockSpec((1, 1, _TILE_V), lambda j: (j, 0, 0)),
        ],
        out_specs=pl.BlockSpec((_TILE_V, BATCH), lambda j: (j, 0)),
        out_shape=jax.ShapeDtypeStruct((VOCAB, BATCH), jnp.float32),
    )(hidden, lm_w, lm_b3d)


def kernel(input_ids, embed_table, lm_w, lm_b):
    hidden = _make_sc_gather()(embed_table, input_ids.astype(jnp.int32))
    lm_b_pad = jnp.pad(lm_b, (0, _NBLK * _TILE_V - VOCAB))
    logits_t = _projection(hidden, lm_w, lm_b_pad.reshape(_NBLK, 1, _TILE_V))
    return logits_t.T


# speedup vs baseline: 3.7155x; 1.0195x over previous
import functools

import jax
import jax.numpy as jnp
from jax import lax
from jax.experimental import pallas as pl
from jax.experimental.pallas import tpu as pltpu
from jax.experimental.pallas import tpu_sc as plsc

VOCAB = 100000
HIDDEN = 128
BATCH = 1024


_NC, _NS = 2, 16
_NW = _NC * _NS
_B_PER_W = BATCH // _NW


@functools.cache
def _make_sc_gather():
    mesh = plsc.VectorSubcoreMesh(core_axis_name="c", subcore_axis_name="s")

    @functools.partial(
        pl.kernel,
        mesh=mesh,
        out_type=jax.ShapeDtypeStruct((BATCH, HIDDEN), jnp.float32),
        scratch_types=[
            pltpu.VMEM((_B_PER_W,), jnp.int32),
            pltpu.VMEM((_B_PER_W, HIDDEN), jnp.float32),
            pltpu.SemaphoreType.DMA,
        ],
    )
    def sc_gather(table_hbm, idx_hbm, out_hbm, idx_v, rows_v, sem):
        wid = lax.axis_index("s") * _NC + lax.axis_index("c")
        base = wid * _B_PER_W
        pltpu.sync_copy(idx_hbm.at[pl.ds(base, _B_PER_W)], idx_v)
        pltpu.async_copy(table_hbm.at[idx_v], rows_v, sem).wait()
        pltpu.sync_copy(rows_v, out_hbm.at[pl.ds(base, _B_PER_W)])

    return sc_gather


_TILE_V = 4096
_NBLK = pl.cdiv(VOCAB, _TILE_V)


def _proj_body(hidden_ref, w_ref, b_ref, out_ref):
    acc = lax.dot_general(
        w_ref[...], hidden_ref[...],
        dimension_numbers=(((1,), (1,)), ((), ())),
        preferred_element_type=jnp.float32,
    )
    out_ref[...] = acc + jnp.transpose(b_ref[...][None, :])


def _projection(hidden, lm_w, lm_b):
    return pl.pallas_call(
        _proj_body,
        grid=(_NBLK,),
        in_specs=[
            pl.BlockSpec((BATCH, HIDDEN), lambda j: (0, 0)),
            pl.BlockSpec((_TILE_V, HIDDEN), lambda j: (j, 0)),
            pl.BlockSpec((_TILE_V,), lambda j: (j,)),
        ],
        out_specs=pl.BlockSpec((_TILE_V, BATCH), lambda j: (j, 0)),
        out_shape=jax.ShapeDtypeStruct((VOCAB, BATCH), jnp.float32),
    )(hidden, lm_w, lm_b)


def kernel(input_ids, embed_table, lm_w, lm_b):
    hidden = _make_sc_gather()(embed_table, input_ids.astype(jnp.int32))
    logits_t = _projection(hidden, lm_w, lm_b)
    return logits_t.T


# speedup vs baseline: 3.7179x; 1.0006x over previous
import functools

import jax
import jax.numpy as jnp
from jax import lax
from jax.experimental import pallas as pl
from jax.experimental.pallas import tpu as pltpu
from jax.experimental.pallas import tpu_sc as plsc

VOCAB = 100000
HIDDEN = 128
BATCH = 1024


_NC, _NS = 2, 16
_NW = _NC * _NS
_B_PER_W = BATCH // _NW


@functools.cache
def _make_sc_gather():
    mesh = plsc.VectorSubcoreMesh(core_axis_name="c", subcore_axis_name="s")

    @functools.partial(
        pl.kernel,
        mesh=mesh,
        out_type=jax.ShapeDtypeStruct((BATCH, HIDDEN), jnp.float32),
        scratch_types=[
            pltpu.VMEM((_B_PER_W,), jnp.int32),
            pltpu.VMEM((_B_PER_W, HIDDEN), jnp.float32),
            pltpu.SemaphoreType.DMA,
        ],
    )
    def sc_gather(table_hbm, idx_hbm, out_hbm, idx_v, rows_v, sem):
        wid = lax.axis_index("s") * _NC + lax.axis_index("c")
        base = wid * _B_PER_W
        pltpu.sync_copy(idx_hbm.at[pl.ds(base, _B_PER_W)], idx_v)
        pltpu.async_copy(table_hbm.at[idx_v], rows_v, sem).wait()
        pltpu.sync_copy(rows_v, out_hbm.at[pl.ds(base, _B_PER_W)])

    return sc_gather


_TILE_V = 4096
_NBLK = pl.cdiv(VOCAB, _TILE_V)


def _proj_body(hidden_ref, w_ref, b_ref, out_ref):
    acc = lax.dot_general(
        w_ref[...], hidden_ref[...],
        dimension_numbers=(((1,), (1,)), ((), ())),
        preferred_element_type=jnp.float32,
    )
    out_ref[...] = acc + jnp.transpose(b_ref[...][None, :])


def _projection(hidden, lm_w, lm_b):
    return pl.pallas_call(
        _proj_body,
        grid=(_NBLK,),
        in_specs=[
            pl.BlockSpec((BATCH, HIDDEN), lambda j: (0, 0)),
            pl.BlockSpec((_TILE_V, HIDDEN), lambda j: (j, 0)),
            pl.BlockSpec((_TILE_V,), lambda j: (j,)),
        ],
        out_specs=pl.BlockSpec((_TILE_V, BATCH), lambda j: (j, 0)),
        out_shape=jax.ShapeDtypeStruct((VOCAB, BATCH), jnp.float32),
    )(hidden, lm_w, lm_b)


def kernel(input_ids, embed_table, lm_w, lm_b):
    hidden = jnp.take(embed_table, input_ids, axis=0)
    logits_t = _projection(hidden, lm_w, lm_b)
    return logits_t.T


# speedup vs baseline: 3.7315x; 1.0037x over previous
import functools

import jax
import jax.numpy as jnp
from jax import lax
from jax.experimental import pallas as pl
from jax.experimental.pallas import tpu as pltpu
from jax.experimental.pallas import tpu_sc as plsc

VOCAB = 100000
HIDDEN = 128
BATCH = 1024


_NC, _NS = 2, 16
_NW = _NC * _NS
_B_PER_W = BATCH // _NW


@functools.cache
def _make_sc_gather():
    mesh = plsc.VectorSubcoreMesh(core_axis_name="c", subcore_axis_name="s")

    @functools.partial(
        pl.kernel,
        mesh=mesh,
        out_type=jax.ShapeDtypeStruct((BATCH, HIDDEN), jnp.float32),
        scratch_types=[
            pltpu.VMEM((_B_PER_W,), jnp.int32),
            pltpu.VMEM((_B_PER_W, HIDDEN), jnp.float32),
            pltpu.SemaphoreType.DMA,
        ],
    )
    def sc_gather(table_hbm, idx_hbm, out_hbm, idx_v, rows_v, sem):
        wid = lax.axis_index("s") * _NC + lax.axis_index("c")
        base = wid * _B_PER_W
        pltpu.sync_copy(idx_hbm.at[pl.ds(base, _B_PER_W)], idx_v)
        pltpu.async_copy(table_hbm.at[idx_v], rows_v, sem).wait()
        pltpu.sync_copy(rows_v, out_hbm.at[pl.ds(base, _B_PER_W)])

    return sc_gather


_TILE_V = 5120
_NBLK = pl.cdiv(VOCAB, _TILE_V)


def _proj_body(hidden_ref, w_ref, b_ref, out_ref):
    acc = lax.dot_general(
        w_ref[...], hidden_ref[...],
        dimension_numbers=(((1,), (1,)), ((), ())),
        preferred_element_type=jnp.float32,
    )
    out_ref[...] = acc + jnp.transpose(b_ref[...][None, :])


def _projection(hidden, lm_w, lm_b):
    return pl.pallas_call(
        _proj_body,
        grid=(_NBLK,),
        in_specs=[
            pl.BlockSpec((BATCH, HIDDEN), lambda j: (0, 0)),
            pl.BlockSpec((_TILE_V, HIDDEN), lambda j: (j, 0)),
            pl.BlockSpec((_TILE_V,), lambda j: (j,)),
        ],
        out_specs=pl.BlockSpec((_TILE_V, BATCH), lambda j: (j, 0)),
        out_shape=jax.ShapeDtypeStruct((VOCAB, BATCH), jnp.float32),
    )(hidden, lm_w, lm_b)


def kernel(input_ids, embed_table, lm_w, lm_b):
    hidden = _make_sc_gather()(embed_table, input_ids.astype(jnp.int32))
    logits_t = _projection(hidden, lm_w, lm_b)
    return logits_t.T


# speedup vs baseline: 3.7343x; 1.0007x over previous
import functools

import jax
import jax.numpy as jnp
from jax import lax
from jax.experimental import pallas as pl
from jax.experimental.pallas import tpu as pltpu
from jax.experimental.pallas import tpu_sc as plsc

VOCAB = 100000
HIDDEN = 128
BATCH = 1024


_NC, _NS = 2, 16
_NW = _NC * _NS
_B_PER_W = BATCH // _NW


@functools.cache
def _make_sc_gather():
    mesh = plsc.VectorSubcoreMesh(core_axis_name="c", subcore_axis_name="s")

    half = _B_PER_W // 2

    @functools.partial(
        pl.kernel,
        mesh=mesh,
        out_type=jax.ShapeDtypeStruct((BATCH, HIDDEN), jnp.float32),
        scratch_types=[
            pltpu.VMEM((_B_PER_W,), jnp.int32),
            pltpu.VMEM((half, HIDDEN), jnp.float32),
            pltpu.VMEM((half, HIDDEN), jnp.float32),
            pltpu.SemaphoreType.DMA,
            pltpu.SemaphoreType.DMA,
            pltpu.SemaphoreType.DMA,
            pltpu.SemaphoreType.DMA,
        ],
    )
    def sc_gather(table_hbm, idx_hbm, out_hbm, idx_v, rows0, rows1,
                  g0, g1, o0, o1):
        wid = lax.axis_index("s") * _NC + lax.axis_index("c")
        base = wid * _B_PER_W
        pltpu.sync_copy(idx_hbm.at[pl.ds(base, _B_PER_W)], idx_v)
        c0 = pltpu.async_copy(table_hbm.at[idx_v.at[pl.ds(0, half)]], rows0, g0)
        c1 = pltpu.async_copy(table_hbm.at[idx_v.at[pl.ds(half, half)]], rows1, g1)
        c0.wait()
        w0 = pltpu.async_copy(rows0, out_hbm.at[pl.ds(base, half)], o0)
        c1.wait()
        w1 = pltpu.async_copy(rows1, out_hbm.at[pl.ds(base + half, half)], o1)
        w0.wait()
        w1.wait()

    return sc_gather


_TILE_V = 5120
_NBLK = pl.cdiv(VOCAB, _TILE_V)


def _proj_body(hidden_ref, w_ref, b_ref, out_ref):
    acc = lax.dot_general(
        w_ref[...], hidden_ref[...],
        dimension_numbers=(((1,), (1,)), ((), ())),
        preferred_element_type=jnp.float32,
    )
    out_ref[...] = acc + jnp.transpose(b_ref[...][None, :])


def _projection(hidden, lm_w, lm_b):
    return pl.pallas_call(
        _proj_body,
        grid=(_NBLK,),
        in_specs=[
            pl.BlockSpec((BATCH, HIDDEN), lambda j: (0, 0)),
            pl.BlockSpec((_TILE_V, HIDDEN), lambda j: (j, 0)),
            pl.BlockSpec((_TILE_V,), lambda j: (j,)),
        ],
        out_specs=pl.BlockSpec((_TILE_V, BATCH), lambda j: (j, 0)),
        out_shape=jax.ShapeDtypeStruct((VOCAB, BATCH), jnp.float32),
    )(hidden, lm_w, lm_b)


def kernel(input_ids, embed_table, lm_w, lm_b):
    hidden = _make_sc_gather()(embed_table, input_ids)
    logits_t = _projection(hidden, lm_w, lm_b)
    return logits_t.T
